# baseline (speedup 1.0000x reference)
import functools

import jax
import jax.numpy as jnp
from jax import lax
from jax.experimental import pallas as pl
from jax.experimental.pallas import tpu as pltpu
from jax.experimental.pallas import tpu_sc as plsc

_EPS = 1e-09
_C = 10
_NW = 32


@functools.cache
def _make_sc(B):
    rows = B // _NW
    outw = rows * _C
    mesh = plsc.VectorSubcoreMesh(core_axis_name="c", subcore_axis_name="s")

    @functools.partial(
        pl.kernel,
        out_type=[
            jax.ShapeDtypeStruct((B * _C,), jnp.float32),
            jax.ShapeDtypeStruct((B * _C,), jnp.float32),
        ],
        mesh=mesh,
        compiler_params=pltpu.CompilerParams(
            needs_layout_passes=False,
            skip_device_barrier=True,
            disable_bounds_checks=True,
            disable_semaphore_checks=True,
        ),
        scratch_types=[
            pltpu.VMEM((rows,), jnp.int32),
            pltpu.VMEM((outw,), jnp.float32),
            pltpu.SemaphoreType.DMA,
        ],
    )
    def k(labels_hbm, mu_hbm, std_hbm, lab_v, buf_v, sem):
        wid = lax.axis_index("s") * 2 + lax.axis_index("c")
        rbase = wid * rows
        obase = wid * outw

        cp = pltpu.async_copy(labels_hbm.at[pl.ds(rbase, rows)], lab_v, sem)

        eps16 = jnp.full((16,), _EPS, jnp.float32)

        def fill(i, carry):
            for j in range(8):
                buf_v[pl.ds((i * 8 + j) * 16, 16)] = eps16
            return carry

        lax.fori_loop(0, outw // 128, fill, 0)
        pltpu.sync_copy(buf_v, std_hbm.at[pl.ds(obase, outw)])

        cp.wait()
        one16 = jnp.full((16,), jnp.float32(1.0 - _EPS), jnp.float32)

        def scat(i, carry):
            lab = lab_v[pl.ds(i * 16, 16)]
            r = lax.iota(jnp.int32, 16) + i * 16
            plsc.store_scatter(buf_v, [r * _C + lab], one16)
            return carry

        lax.fori_loop(0, rows // 16, scat, 0)
        pltpu.sync_copy(buf_v, mu_hbm.at[pl.ds(obase, outw)])

    return k


def kernel(labels, cuda):
    B = labels.shape[0]
    mu, std = _make_sc(B)(labels)
    return mu.reshape(1, B, _C), std.reshape(1, B, _C)


# speedup vs baseline: 1.0206x; 1.0206x over previous
import functools

import jax
import jax.numpy as jnp
from jax import lax
from jax.experimental import pallas as pl
from jax.experimental.pallas import tpu as pltpu
from jax.experimental.pallas import tpu_sc as plsc

_EPS = 1e-09
_C = 10
_NW = 16


@functools.cache
def _make_sc(B):
    rows = B // _NW
    outw = rows * _C
    mesh = plsc.VectorSubcoreMesh(
        core_axis_name="c", subcore_axis_name="s", num_cores=1
    )

    @functools.partial(
        pl.kernel,
        out_type=[
            jax.ShapeDtypeStruct((B * _C,), jnp.float32),
            jax.ShapeDtypeStruct((B * _C,), jnp.float32),
        ],
        mesh=mesh,
        compiler_params=pltpu.CompilerParams(
            needs_layout_passes=False,
            skip_device_barrier=True,
            disable_bounds_checks=True,
            disable_semaphore_checks=True,
        ),
        scratch_types=[
            pltpu.VMEM((rows,), jnp.int32),
            pltpu.VMEM((outw,), jnp.float32),
            pltpu.SemaphoreType.DMA,
        ],
    )
    def k(labels_hbm, mu_hbm, std_hbm, lab_v, buf_v, sem):
        wid = lax.axis_index("s") + lax.axis_index("c") * 16
        rbase = wid * rows
        obase = wid * outw

        cp = pltpu.async_copy(labels_hbm.at[pl.ds(rbase, rows)], lab_v, sem)

        eps16 = jnp.full((16,), _EPS, jnp.float32)

        def fill(i, carry):
            for j in range(8):
                buf_v[pl.ds((i * 8 + j) * 16, 16)] = eps16
            return carry

        lax.fori_loop(0, outw // 128, fill, 0)
        pltpu.sync_copy(buf_v, std_hbm.at[pl.ds(obase, outw)])

        cp.wait()
        one16 = jnp.full((16,), jnp.float32(1.0 - _EPS), jnp.float32)

        def scat(i, carry):
            lab = lab_v[pl.ds(i * 16, 16)]
            r = lax.iota(jnp.int32, 16) + i * 16
            plsc.store_scatter(buf_v, [r * _C + lab], one16)
            return carry

        lax.fori_loop(0, rows // 16, scat, 0)
        pltpu.sync_copy(buf_v, mu_hbm.at[pl.ds(obase, outw)])

    return k


def kernel(labels, cuda):
    B = labels.shape[0]
    mu, std = _make_sc(B)(labels)
    return mu.reshape(1, B, _C), std.reshape(1, B, _C)


# speedup vs baseline: 1.3938x; 1.3658x over previous
import functools

import jax
import jax.numpy as jnp
from jax import lax
from jax.experimental import pallas as pl
from jax.experimental.pallas import tpu as pltpu
from jax.experimental.pallas import tpu_sc as plsc

_EPS = 1e-09
_C = 10
_NW = 32


@functools.cache
def _make_sc(B):
    rows = B // _NW
    mesh = plsc.VectorSubcoreMesh(core_axis_name="c", subcore_axis_name="s")

    @functools.partial(
        pl.kernel,
        out_type=[
            jax.ShapeDtypeStruct((B, _C), jnp.float32),
            jax.ShapeDtypeStruct((B, _C), jnp.float32),
        ],
        mesh=mesh,
        compiler_params=pltpu.CompilerParams(
            needs_layout_passes=False,
            use_tc_tiling_on_sc=True,
        ),
        scratch_types=[
            pltpu.VMEM((rows,), jnp.int32),
            pltpu.VMEM((rows, _C), jnp.float32),
            pltpu.SemaphoreType.DMA,
        ],
    )
    def k(labels_hbm, mu_hbm, std_hbm, lab_v, buf_v, sem):
        wid = lax.axis_index("s") * 2 + lax.axis_index("c")
        rbase = wid * rows

        cp = pltpu.async_copy(labels_hbm.at[pl.ds(rbase, rows)], lab_v, sem)

        eps16 = jnp.full((16,), _EPS, jnp.float32)
        iota16 = lax.iota(jnp.int32, 16)

        def fill(i, carry):
            r = iota16 + i * 16
            for j in range(_C):
                plsc.store_scatter(buf_v, [r, jnp.full((16,), j, jnp.int32)], eps16)
            return carry

        lax.fori_loop(0, rows // 16, fill, 0)
        pltpu.sync_copy(buf_v, std_hbm.at[pl.ds(rbase, rows)])

        cp.wait()
        one16 = jnp.full((16,), jnp.float32(1.0 - _EPS), jnp.float32)

        def scat(i, carry):
            lab = lab_v[pl.ds(i * 16, 16)]
            r = iota16 + i * 16
            plsc.store_scatter(buf_v, [r, lab], one16)
            return carry

        lax.fori_loop(0, rows // 16, scat, 0)
        pltpu.sync_copy(buf_v, mu_hbm.at[pl.ds(rbase, rows)])

    return k


def kernel(labels, cuda):
    B = labels.shape[0]
    mu, std = _make_sc(B)(labels)
    return mu.reshape(1, B, _C), std.reshape(1, B, _C)


# speedup vs baseline: 2.6511x; 1.9020x over previous
import functools

import jax
import jax.numpy as jnp
from jax import lax
from jax.experimental import pallas as pl
from jax.experimental.pallas import tpu as pltpu
from jax.experimental.pallas import tpu_sc as plsc

_EPS = 1e-09
_C = 10
_NW = 32


@functools.cache
def _make_sc(B):
    cols = B // _NW
    mesh = plsc.VectorSubcoreMesh(core_axis_name="c", subcore_axis_name="s")

    @functools.partial(
        pl.kernel,
        out_type=[
            jax.ShapeDtypeStruct((_C * B,), jnp.float32),
            jax.ShapeDtypeStruct((_C * B,), jnp.float32),
        ],
        mesh=mesh,
        compiler_params=pltpu.CompilerParams(needs_layout_passes=False),
        scratch_types=[
            pltpu.VMEM((cols,), jnp.int32),
            pltpu.VMEM((_C * cols,), jnp.float32),
            pltpu.SemaphoreType.DMA,
            pltpu.SemaphoreType.DMA,
        ],
    )
    def k(labels_hbm, mu_hbm, std_hbm, lab_v, buf_v, lsem, osem):
        wid = lax.axis_index("s") * 2 + lax.axis_index("c")
        cbase = wid * cols

        cp = pltpu.async_copy(labels_hbm.at[pl.ds(cbase, cols)], lab_v, lsem)

        eps16 = jnp.full((16,), _EPS, jnp.float32)

        def fill(i, carry):
            for j in range(_C):
                buf_v[pl.ds((i * _C + j) * 16, 16)] = eps16
            return carry

        lax.fori_loop(0, (_C * cols) // (16 * _C), fill, 0)

        std_cps = [
            pltpu.async_copy(
                buf_v.at[pl.ds(c * cols, cols)],
                std_hbm.at[pl.ds(c * B + cbase, cols)],
                osem,
            )
            for c in range(_C)
        ]
        cp.wait()
        for scp in std_cps:
            scp.wait()

        one16 = jnp.full((16,), jnp.float32(1.0 - _EPS), jnp.float32)
        iota16 = lax.iota(jnp.int32, 16)

        def scat(i, carry):
            lab = lab_v[pl.ds(i * 16, 16)]
            col = iota16 + i * 16
            plsc.store_scatter(buf_v, [lab * cols + col], one16)
            return carry

        lax.fori_loop(0, cols // 16, scat, 0)

        mu_cps = [
            pltpu.async_copy(
                buf_v.at[pl.ds(c * cols, cols)],
                mu_hbm.at[pl.ds(c * B + cbase, cols)],
                osem,
            )
            for c in range(_C)
        ]
        for mcp in mu_cps:
            mcp.wait()

    return k


def kernel(labels, cuda):
    B = labels.shape[0]
    mu_f, std_f = _make_sc(B)(labels)
    mu = jnp.transpose(mu_f.reshape(1, _C, B), (0, 2, 1))
    std = jnp.transpose(std_f.reshape(1, _C, B), (0, 2, 1))
    return mu, std


# speedup vs baseline: 2.6520x; 1.0004x over previous
import functools

import jax
import jax.numpy as jnp
from jax import lax
from jax.experimental import pallas as pl
from jax.experimental.pallas import tpu as pltpu
from jax.experimental.pallas import tpu_sc as plsc

_EPS = 1e-09
_C = 10
_NW = 32


@functools.cache
def _make_sc(B):
    cols = B // _NW
    mesh = plsc.VectorSubcoreMesh(core_axis_name="c", subcore_axis_name="s")

    @functools.partial(
        pl.kernel,
        out_type=[
            jax.ShapeDtypeStruct((_C * B,), jnp.float32),
            jax.ShapeDtypeStruct((_C * B,), jnp.float32),
        ],
        mesh=mesh,
        compiler_params=pltpu.CompilerParams(
            needs_layout_passes=False,
            skip_device_barrier=True,
        ),
        scratch_types=[
            pltpu.VMEM((cols,), jnp.int32),
            pltpu.VMEM((_C * cols,), jnp.float32),
            pltpu.SemaphoreType.DMA,
            pltpu.SemaphoreType.DMA,
        ],
    )
    def k(labels_hbm, mu_hbm, std_hbm, lab_v, buf_v, lsem, osem):
        wid = lax.axis_index("s") * 2 + lax.axis_index("c")
        cbase = wid * cols

        cp = pltpu.async_copy(labels_hbm.at[pl.ds(cbase, cols)], lab_v, lsem)

        eps16 = jnp.full((16,), _EPS, jnp.float32)

        def fill(i, carry):
            for j in range(_C):
                buf_v[pl.ds((i * _C + j) * 16, 16)] = eps16
            return carry

        lax.fori_loop(0, (_C * cols) // (16 * _C), fill, 0)

        std_cps = [
            pltpu.async_copy(
                buf_v.at[pl.ds(c * cols, cols)],
                std_hbm.at[pl.ds(c * B + cbase, cols)],
                osem,
            )
            for c in range(_C)
        ]
        cp.wait()
        for scp in std_cps:
            scp.wait()

        one16 = jnp.full((16,), jnp.float32(1.0 - _EPS), jnp.float32)
        iota16 = lax.iota(jnp.int32, 16)

        def scat(i, carry):
            lab = lab_v[pl.ds(i * 16, 16)]
            col = iota16 + i * 16
            plsc.store_scatter(buf_v, [lab * cols + col], one16)
            return carry

        lax.fori_loop(0, cols // 16, scat, 0)

        mu_cps = [
            pltpu.async_copy(
                buf_v.at[pl.ds(c * cols, cols)],
                mu_hbm.at[pl.ds(c * B + cbase, cols)],
                osem,
            )
            for c in range(_C)
        ]
        for mcp in mu_cps:
            mcp.wait()

    return k


def kernel(labels, cuda):
    B = labels.shape[0]
    mu_f, std_f = _make_sc(B)(labels)
    mu = jnp.transpose(mu_f.reshape(1, _C, B), (0, 2, 1))
    std = jnp.transpose(std_f.reshape(1, _C, B), (0, 2, 1))
    return mu, std


# speedup vs baseline: 2.7460x; 1.0354x over previous
import functools

import jax
import jax.numpy as jnp
from jax import lax
from jax.experimental import pallas as pl
from jax.experimental.pallas import tpu as pltpu
from jax.experimental.pallas import tpu_sc as plsc

_EPS = 1e-09
_C = 10
_NW = 16


@functools.cache
def _make_sc(B):
    cols = B // _NW
    mesh = plsc.VectorSubcoreMesh(
        core_axis_name="c", subcore_axis_name="s", num_cores=1
    )

    @functools.partial(
        pl.kernel,
        out_type=[
            jax.ShapeDtypeStruct((_C * B,), jnp.float32),
            jax.ShapeDtypeStruct((_C * B,), jnp.float32),
        ],
        mesh=mesh,
        compiler_params=pltpu.CompilerParams(
            needs_layout_passes=False,
            skip_device_barrier=True,
        ),
        scratch_types=[
            pltpu.VMEM((cols,), jnp.int32),
            pltpu.VMEM((_C * cols,), jnp.float32),
            pltpu.SemaphoreType.DMA,
            pltpu.SemaphoreType.DMA,
        ],
    )
    def k(labels_hbm, mu_hbm, std_hbm, lab_v, buf_v, lsem, osem):
        wid = lax.axis_index("s") + lax.axis_index("c") * 16
        cbase = wid * cols

        cp = pltpu.async_copy(labels_hbm.at[pl.ds(cbase, cols)], lab_v, lsem)

        eps16 = jnp.full((16,), _EPS, jnp.float32)

        def fill(i, carry):
            for j in range(_C):
                buf_v[pl.ds((i * _C + j) * 16, 16)] = eps16
            return carry

        lax.fori_loop(0, (_C * cols) // (16 * _C), fill, 0)

        std_cps = [
            pltpu.async_copy(
                buf_v.at[pl.ds(c * cols, cols)],
                std_hbm.at[pl.ds(c * B + cbase, cols)],
                osem,
            )
            for c in range(_C)
        ]
        cp.wait()
        for scp in std_cps:
            scp.wait()

        one16 = jnp.full((16,), jnp.float32(1.0 - _EPS), jnp.float32)
        iota16 = lax.iota(jnp.int32, 16)

        def scat(i, carry):
            lab = lab_v[pl.ds(i * 16, 16)]
            col = iota16 + i * 16
            plsc.store_scatter(buf_v, [lab * cols + col], one16)
            return carry

        lax.fori_loop(0, cols // 16, scat, 0)

        mu_cps = [
            pltpu.async_copy(
                buf_v.at[pl.ds(c * cols, cols)],
                mu_hbm.at[pl.ds(c * B + cbase, cols)],
                osem,
            )
            for c in range(_C)
        ]
        for mcp in mu_cps:
            mcp.wait()

    return k


def kernel(labels, cuda):
    B = labels.shape[0]
    mu_f, std_f = _make_sc(B)(labels)
    mu = jnp.transpose(mu_f.reshape(1, _C, B), (0, 2, 1))
    std = jnp.transpose(std_f.reshape(1, _C, B), (0, 2, 1))
    return mu, std


# speedup vs baseline: 11.7507x; 4.2792x over previous
import functools

import jax
import jax.numpy as jnp
from jax.experimental import pallas as pl

_EPS = 1e-09
_C = 10


@functools.cache
def _make_tc(B):
    def body(lab_ref, mu_ref, std_ref):
        c = pl.program_id(0)
        lab = lab_ref[...]
        mu_ref[...] = jnp.where(
            lab == c, jnp.float32(1.0 - _EPS), jnp.float32(_EPS)
        )
        std_ref[...] = jnp.full((B,), _EPS, jnp.float32)

    return pl.pallas_call(
        body,
        grid=(_C,),
        in_specs=[pl.BlockSpec((B,), lambda c: (0,))],
        out_specs=[
            pl.BlockSpec((B,), lambda c: (c,)),
            pl.BlockSpec((B,), lambda c: (c,)),
        ],
        out_shape=[
            jax.ShapeDtypeStruct((_C * B,), jnp.float32),
            jax.ShapeDtypeStruct((_C * B,), jnp.float32),
        ],
    )


def kernel(labels, cuda):
    B = labels.shape[0]
    mu_f, std_f = _make_tc(B)(labels)
    mu = jnp.transpose(mu_f.reshape(1, _C, B), (0, 2, 1))
    std = jnp.transpose(std_f.reshape(1, _C, B), (0, 2, 1))
    return mu, std
